# baseline (speedup 1.0000x reference)
import functools

import jax
import jax.numpy as jnp
from jax import lax
from jax.experimental import pallas as pl
from jax.experimental.pallas import tpu as pltpu
from jax.experimental.pallas import tpu_sc as plsc

N_NODES = 100000
MAX_DEGREE = 64
BATCH = 16384
N_SAMPLES = 25

NC = 2
NS = 16
NW = NC * NS
BPW = BATCH // NW
CHUNK = 128
NCHUNK = BPW // CHUNK
PAIRW = 2 * MAX_DEGREE
OPW = BPW * N_SAMPLES
VPC = BPW // 16
UNROLL = 8

_mesh = plsc.VectorSubcoreMesh(core_axis_name="c", subcore_axis_name="s")


@functools.partial(
    pl.kernel,
    mesh=_mesh,
    compiler_params=pltpu.CompilerParams(
        needs_layout_passes=False, use_tc_tiling_on_sc=True),
    out_type=jax.ShapeDtypeStruct((N_SAMPLES, BATCH), jnp.int32),
    scratch_types=[
        pltpu.VMEM((BPW,), jnp.int32),
        pltpu.VMEM((8, CHUNK), jnp.int32),
        pltpu.VMEM((BPW,), jnp.int32),
        pltpu.VMEM((BPW, PAIRW), jnp.int32),
        pltpu.VMEM((32, BPW), jnp.int32),
        pltpu.VMEM((32,), jnp.int32),
        pltpu.SemaphoreType.DMA,
        pltpu.SemaphoreType.DMA,
    ],
)
def _sample_one_table(ids_hbm, cols_hbm, tbl_hbm, out_hbm,
                      idx_v, gidx_v, par_v, rows_v, out_v, cols_v,
                      sem, sem_out):
    wid = lax.axis_index("s") * NC + lax.axis_index("c")

    pltpu.sync_copy(cols_hbm, cols_v)
    pltpu.sync_copy(ids_hbm.at[pl.ds(wid * BPW, BPW)], idx_v)
    for c in range(NCHUNK):
        for k in range(CHUNK // 16):
            s = pl.ds(c * CHUNK + k * 16, 16)
            v = idx_v[s]
            gidx_v[c, pl.ds(k * 16, 16)] = v >> 1
            par_v[s] = (v & 1) << 6

    copies = []
    for c in range(NCHUNK):
        dst = pl.ds(c * CHUNK, CHUNK)
        copies.append(
            pltpu.async_copy(tbl_hbm.at[gidx_v.at[c]], rows_v.at[dst], sem))
    for cp in copies:
        cp.wait()

    lanes = lax.iota(jnp.int32, 16)
    cols_lo = cols_v[pl.ds(0, 16)]
    cols_hi = cols_v[pl.ds(16, 16)]

    for j in range(N_SAMPLES):
        cj = cols_lo[j] if j < 16 else cols_hi[j - 16]

        def vec_body(i, carry2, cj=cj, j=j):
            for u in range(UNROLL):
                s = pl.ds(i * (UNROLL * 16) + u * 16, 16)
                r = lanes + i * (UNROLL * 16) + u * 16
                c = par_v[s] + cj
                out_v[j, s] = plsc.load_gather(rows_v, [r, c])
            return carry2

        lax.fori_loop(0, VPC // UNROLL, vec_body, 0)
        pltpu.async_copy(
            out_v.at[pl.ds(j, 1)],
            out_hbm.at[pl.ds(j, 1), pl.ds(wid * BPW, BPW)], sem_out)

    def drain(j, carry):
        pltpu.make_async_copy(
            out_v.at[pl.ds(0, 1)],
            out_hbm.at[pl.ds(0, 1), pl.ds(wid * BPW, BPW)], sem_out).wait()
        return carry

    lax.fori_loop(0, N_SAMPLES, drain, 0)


def kernel(ids, num_samples, adj_info, adj_answer):
    perm = jax.random.permutation(jax.random.key(42), MAX_DEGREE)
    start = jnp.asarray(num_samples, jnp.int32) - N_SAMPLES
    cols = lax.dynamic_slice(perm, (start,), (N_SAMPLES,)).astype(jnp.int32)
    cols32 = jnp.concatenate([cols, jnp.zeros((32 - N_SAMPLES,), jnp.int32)])

    ids32 = ids.astype(jnp.int32)
    o_info = _sample_one_table(
        ids32, cols32, adj_info.reshape(N_NODES // 2, PAIRW))
    o_ans = _sample_one_table(
        ids32, cols32, adj_answer.reshape(N_NODES // 2, PAIRW))
    return (o_info.T, o_ans.T)


# speedup vs baseline: 1.0005x; 1.0005x over previous
import functools

import jax
import jax.numpy as jnp
from jax import lax
from jax.experimental import pallas as pl
from jax.experimental.pallas import tpu as pltpu
from jax.experimental.pallas import tpu_sc as plsc

N_NODES = 100000
MAX_DEGREE = 64
BATCH = 16384
N_SAMPLES = 25

NC = 2
NS = 16
NW = NC * NS
BPW = BATCH // NW
CHUNK = 128
NCHUNK = BPW // CHUNK
PAIRW = 2 * MAX_DEGREE
OPW = BPW * N_SAMPLES
VPC = BPW // 16
UNROLL = 4

_mesh = plsc.VectorSubcoreMesh(core_axis_name="c", subcore_axis_name="s")


@functools.partial(
    pl.kernel,
    mesh=_mesh,
    compiler_params=pltpu.CompilerParams(
        needs_layout_passes=False, use_tc_tiling_on_sc=True),
    out_type=jax.ShapeDtypeStruct((N_SAMPLES, BATCH), jnp.int32),
    scratch_types=[
        pltpu.VMEM((BPW,), jnp.int32),
        pltpu.VMEM((8, CHUNK), jnp.int32),
        pltpu.VMEM((BPW,), jnp.int32),
        pltpu.VMEM((BPW, PAIRW), jnp.int32),
        pltpu.VMEM((32, BPW), jnp.int32),
        pltpu.VMEM((32,), jnp.int32),
        pltpu.SemaphoreType.DMA,
        pltpu.SemaphoreType.DMA,
    ],
)
def _sample_one_table(ids_hbm, cols_hbm, tbl_hbm, out_hbm,
                      idx_v, gidx_v, par_v, rows_v, out_v, cols_v,
                      sem, sem_out):
    wid = lax.axis_index("s") * NC + lax.axis_index("c")

    pltpu.sync_copy(cols_hbm, cols_v)
    pltpu.sync_copy(ids_hbm.at[pl.ds(wid * BPW, BPW)], idx_v)
    for c in range(NCHUNK):
        for k in range(CHUNK // 16):
            s = pl.ds(c * CHUNK + k * 16, 16)
            v = idx_v[s]
            gidx_v[c, pl.ds(k * 16, 16)] = v >> 1
            par_v[s] = (v & 1) << 6

    copies = []
    for c in range(NCHUNK):
        dst = pl.ds(c * CHUNK, CHUNK)
        copies.append(
            pltpu.async_copy(tbl_hbm.at[gidx_v.at[c]], rows_v.at[dst], sem))
    for cp in copies:
        cp.wait()

    lanes = lax.iota(jnp.int32, 16)
    cols_lo = cols_v[pl.ds(0, 16)]
    cols_hi = cols_v[pl.ds(16, 16)]

    for j in range(N_SAMPLES):
        cj = cols_lo[j] if j < 16 else cols_hi[j - 16]

        def vec_body(i, carry2, cj=cj, j=j):
            for u in range(UNROLL):
                s = pl.ds(i * (UNROLL * 16) + u * 16, 16)
                r = lanes + i * (UNROLL * 16) + u * 16
                c = par_v[s] + cj
                out_v[j, s] = plsc.load_gather(rows_v, [r, c])
            return carry2

        lax.fori_loop(0, VPC // UNROLL, vec_body, 0)
        pltpu.async_copy(
            out_v.at[pl.ds(j, 1)],
            out_hbm.at[pl.ds(j, 1), pl.ds(wid * BPW, BPW)], sem_out)

    def drain(j, carry):
        pltpu.make_async_copy(
            out_v.at[pl.ds(0, 1)],
            out_hbm.at[pl.ds(0, 1), pl.ds(wid * BPW, BPW)], sem_out).wait()
        return carry

    lax.fori_loop(0, N_SAMPLES, drain, 0)


def kernel(ids, num_samples, adj_info, adj_answer):
    perm = jax.random.permutation(jax.random.key(42), MAX_DEGREE)
    start = jnp.asarray(num_samples, jnp.int32) - N_SAMPLES
    cols = lax.dynamic_slice(perm, (start,), (N_SAMPLES,)).astype(jnp.int32)
    cols32 = jnp.concatenate([cols, jnp.zeros((32 - N_SAMPLES,), jnp.int32)])

    ids32 = ids.astype(jnp.int32)
    o_info = _sample_one_table(
        ids32, cols32, adj_info.reshape(N_NODES // 2, PAIRW))
    o_ans = _sample_one_table(
        ids32, cols32, adj_answer.reshape(N_NODES // 2, PAIRW))
    return (o_info.T, o_ans.T)
